# Initial kernel scaffold; baseline (speedup 1.0000x reference)
import jax
import jax.numpy as jnp
from jax.experimental import pallas as pl


def kernel(X_d, Y_d, h0_W0, h0_b0, h0_W1, h0_b1, h0_W2, h0_b2, h0_W3, h0_b3, hop0_Wq, hop0_Wk, h1_W0, h1_b0, h1_W1, h1_b1, h1_W2, h1_b2, h1_W3, h1_b3, hop1_Wq, hop1_Wk, h2_W0, h2_b0, h2_W1, h2_b1, h2_W2, h2_b2, h2_W3, h2_b3, hop2_Wq, hop2_Wk, h3_W0, h3_b0, h3_W1, h3_b1, h3_W2, h3_b2, h3_W3, h3_b3, hop3_Wq, hop3_Wk, train):
    raise NotImplementedError("write your pallas kernel here")


# speedup vs baseline: 27.2292x; 27.2292x over previous
import functools

import numpy as np
import jax
import jax.numpy as jnp
from jax.experimental import pallas as pl
from jax.experimental.pallas import tpu as pltpu

MHN = 4
NHEADS = 4
CIN = 128
COUT = 128
B = 64
S = 256
A = 5
K = 20
ALPHAS = (0.05, 0.06, 0.08, 0.1, 0.12)


def _qpos(q):
    loc = q * (K - 1)
    f = int(np.floor(loc))
    return f, float(loc - f)


_LOW = [_qpos(a / 2.0) for a in ALPHAS]
_HIGH = [_qpos(1.0 - a + a / 2.0) for a in ALPHAS]


def _body(x_ref, y_ref, w0, c0, w1, c1, w2, c2, w3, c3, wq, wk,
          ylow_ref, yhigh_ref, sc_ref):
    b = pl.program_id(1)
    x = x_ref[0, 0]
    yrow = y_ref[0, 0]

    h = jnp.maximum(jnp.dot(x, w0[0], preferred_element_type=jnp.float32) + c0[0], 0.0)
    h = jnp.maximum(jnp.dot(h, w1[0], preferred_element_type=jnp.float32) + c1[0], 0.0)
    h = jnp.maximum(jnp.dot(h, w2[0], preferred_element_type=jnp.float32) + c2[0], 0.0)
    enc = jnp.dot(h, w3[0], preferred_element_type=jnp.float32) + c3[0]
    q = jnp.dot(enc, wq[0], preferred_element_type=jnp.float32)
    k = jnp.dot(enc, wk[0], preferred_element_type=jnp.float32)

    iota_m = jax.lax.broadcasted_iota(jnp.int32, (S, S), 0)
    ybc = jnp.broadcast_to(yrow.reshape(S, 1), (S, S))
    inf = jnp.float32(np.inf)
    total = jnp.float32(0.0)

    for hh in range(NHEADS):
        qh = q[:, hh * COUT:(hh + 1) * COUT]
        kh = k[:, hh * COUT:(hh + 1) * COUT]
        cur = jax.lax.dot_general(kh, qh, (((1,), (1,)), ((), ())),
                                  preferred_element_type=jnp.float32)
        m1 = jnp.full((1, S), inf, jnp.float32)
        m2 = jnp.full((1, S), inf, jnp.float32)
        m3 = jnp.full((1, S), inf, jnp.float32)
        b1 = jnp.full((1, S), -inf, jnp.float32)
        b2 = jnp.full((1, S), -inf, jnp.float32)
        b3 = jnp.full((1, S), -inf, jnp.float32)
        for _ in range(K):
            mx = jnp.max(cur, axis=0, keepdims=True)
            wi = jnp.where(cur == mx, iota_m, S)
            mi = jnp.min(wi, axis=0, keepdims=True)
            oh = wi == mi
            yt = jnp.min(jnp.where(oh, ybc, inf), axis=0, keepdims=True)
            cur = jnp.where(oh, -inf, cur)
            t1 = jnp.maximum(m1, yt); m1 = jnp.minimum(m1, yt)
            t2 = jnp.maximum(m2, t1); m2 = jnp.minimum(m2, t1)
            m3 = jnp.minimum(m3, t2)
            u1 = jnp.minimum(b1, yt); b1 = jnp.maximum(b1, yt)
            u2 = jnp.minimum(b2, u1); b2 = jnp.maximum(b2, u1)
            b3 = jnp.maximum(b3, u2)
        lows = []
        for f, g in _LOW:
            lo_v, hi_v = (m1, m2) if f == 0 else (m2, m3)
            lows.append(lo_v * jnp.float32(1.0 - g) + hi_v * jnp.float32(g))
        highs = []
        for f, g in _HIGH:
            lo_v, hi_v = (b2, b1) if f == 18 else (b3, b2)
            highs.append(lo_v * jnp.float32(1.0 - g) + hi_v * jnp.float32(g))
        qlow = jnp.concatenate(lows, axis=0)
        qhigh = jnp.concatenate(highs, axis=0)
        ylow_ref[0, 0, hh] = qlow
        yhigh_ref[0, 0, hh] = qhigh
        for ai, a in enumerate(ALPHAS):
            ql = lows[ai]
            qh2 = highs[ai]
            c = jnp.float32(2.0 / a)
            pen = (jnp.abs(qh2 - ql)
                   + jnp.where(yrow < ql, (ql - yrow) * c, 0.0)
                   + jnp.where(yrow > qh2, (yrow - qh2) * c, 0.0))
            total = total + jnp.sum(pen)

    @pl.when(b == 0)
    def _init():
        sc_ref[...] = jnp.zeros_like(sc_ref)

    sc_ref[...] += total * jnp.float32(1.0 / (NHEADS * A * B * S))


def kernel(X_d, Y_d,
           h0_W0, h0_b0, h0_W1, h0_b1, h0_W2, h0_b2, h0_W3, h0_b3, hop0_Wq, hop0_Wk,
           h1_W0, h1_b0, h1_W1, h1_b1, h1_W2, h1_b2, h1_W3, h1_b3, hop1_Wq, hop1_Wk,
           h2_W0, h2_b0, h2_W1, h2_b1, h2_W2, h2_b2, h2_W3, h2_b3, hop2_Wq, hop2_Wk,
           h3_W0, h3_b0, h3_W1, h3_b1, h3_W2, h3_b2, h3_W3, h3_b3, hop3_Wq, hop3_Wk,
           train=0):
    per_model = [
        [h0_W0, h0_b0, h0_W1, h0_b1, h0_W2, h0_b2, h0_W3, h0_b3, hop0_Wq, hop0_Wk],
        [h1_W0, h1_b0, h1_W1, h1_b1, h1_W2, h1_b2, h1_W3, h1_b3, hop1_Wq, hop1_Wk],
        [h2_W0, h2_b0, h2_W1, h2_b1, h2_W2, h2_b2, h2_W3, h2_b3, hop2_Wq, hop2_Wk],
        [h3_W0, h3_b0, h3_W1, h3_b1, h3_W2, h3_b2, h3_W3, h3_b3, hop3_Wq, hop3_Wk],
    ]
    stacked = []
    for j in range(10):
        arrs = [per_model[i][j] for i in range(MHN)]
        st = jnp.stack(arrs)
        if st.ndim == 2:
            st = st[:, None, :]
        stacked.append(st)

    Y2 = Y_d[:, :, :, 0].reshape(B, MHN, 1, S)

    in_specs = [
        pl.BlockSpec((1, 1, S, CIN), lambda i, b: (b, i, 0, 0)),
        pl.BlockSpec((1, 1, 1, S), lambda i, b: (b, i, 0, 0)),
    ]
    for st in stacked:
        in_specs.append(
            pl.BlockSpec((1,) + st.shape[1:], lambda i, b: (i, 0, 0)))

    out_shape = [
        jax.ShapeDtypeStruct((B, MHN, NHEADS, A, S), jnp.float32),
        jax.ShapeDtypeStruct((B, MHN, NHEADS, A, S), jnp.float32),
        jax.ShapeDtypeStruct((MHN, 8, 128), jnp.float32),
    ]
    out_specs = [
        pl.BlockSpec((1, 1, NHEADS, A, S), lambda i, b: (b, i, 0, 0, 0)),
        pl.BlockSpec((1, 1, NHEADS, A, S), lambda i, b: (b, i, 0, 0, 0)),
        pl.BlockSpec((1, 8, 128), lambda i, b: (i, 0, 0)),
    ]

    ylow_k, yhigh_k, sc_k = pl.pallas_call(
        _body,
        grid=(MHN, B),
        in_specs=in_specs,
        out_specs=out_specs,
        out_shape=out_shape,
        compiler_params=pltpu.CompilerParams(
            dimension_semantics=("arbitrary", "arbitrary")),
    )(X_d, Y2, *stacked)

    scores = sc_k[:, 0, 0] + jnp.asarray(train, jnp.float32) * 0.0
    y = jnp.transpose(Y_d[:, :, :, 0], (0, 2, 1))
    y_low = jnp.transpose(ylow_k, (2, 0, 3, 4, 1))
    y_high = jnp.transpose(yhigh_k, (2, 0, 3, 4, 1))
    return (scores, y, y_low, y_high)


# speedup vs baseline: 48.3117x; 1.7743x over previous
import functools

import numpy as np
import jax
import jax.numpy as jnp
from jax.experimental import pallas as pl
from jax.experimental.pallas import tpu as pltpu

MHN = 4
NHEADS = 4
CIN = 128
COUT = 128
B = 64
S = 256
A = 5
K = 20
ALPHAS = (0.05, 0.06, 0.08, 0.1, 0.12)


def _qpos(q):
    loc = q * (K - 1)
    f = int(np.floor(loc))
    return f, float(loc - f)


_LOW = [_qpos(a / 2.0) for a in ALPHAS]
_HIGH = [_qpos(1.0 - a + a / 2.0) for a in ALPHAS]


def _body(x_ref, y_ref, w0, c0, w1, c1, w2, c2, w3, c3, wq, wk,
          ylow_ref, yhigh_ref, sc_ref):
    b = pl.program_id(1)
    x = x_ref[0, 0]
    yrow = y_ref[0, 0]

    h = jnp.maximum(jnp.dot(x, w0[0], preferred_element_type=jnp.float32) + c0[0], 0.0)
    h = jnp.maximum(jnp.dot(h, w1[0], preferred_element_type=jnp.float32) + c1[0], 0.0)
    h = jnp.maximum(jnp.dot(h, w2[0], preferred_element_type=jnp.float32) + c2[0], 0.0)
    enc = jnp.dot(h, w3[0], preferred_element_type=jnp.float32) + c3[0]
    q = jnp.dot(enc, wq[0], preferred_element_type=jnp.float32)
    k = jnp.dot(enc, wk[0], preferred_element_type=jnp.float32)

    ybc = jnp.broadcast_to(yrow.reshape(S, 1), (S, S))
    inf = jnp.float32(np.inf)
    total = jnp.float32(0.0)

    for hh in range(NHEADS):
        qh = q[:, hh * COUT:(hh + 1) * COUT]
        kh = k[:, hh * COUT:(hh + 1) * COUT]
        cur = jax.lax.dot_general(kh, qh, (((1,), (1,)), ((), ())),
                                  preferred_element_type=jnp.float32)
        for _ in range(K):
            mx = jnp.max(cur, axis=0, keepdims=True)
            cur = jnp.where(cur == mx, -inf, cur)
        sel = cur == -inf
        ylo = jnp.where(sel, ybc, inf)
        yhi = jnp.where(sel, ybc, -inf)
        m1 = jnp.min(ylo, axis=0, keepdims=True)
        ylo = jnp.where(ylo == m1, inf, ylo)
        m2 = jnp.min(ylo, axis=0, keepdims=True)
        ylo = jnp.where(ylo == m2, inf, ylo)
        m3 = jnp.min(ylo, axis=0, keepdims=True)
        b1 = jnp.max(yhi, axis=0, keepdims=True)
        yhi = jnp.where(yhi == b1, -inf, yhi)
        b2 = jnp.max(yhi, axis=0, keepdims=True)
        yhi = jnp.where(yhi == b2, -inf, yhi)
        b3 = jnp.max(yhi, axis=0, keepdims=True)
        lows = []
        for f, g in _LOW:
            lo_v, hi_v = (m1, m2) if f == 0 else (m2, m3)
            lows.append(lo_v * jnp.float32(1.0 - g) + hi_v * jnp.float32(g))
        highs = []
        for f, g in _HIGH:
            lo_v, hi_v = (b2, b1) if f == 18 else (b3, b2)
            highs.append(lo_v * jnp.float32(1.0 - g) + hi_v * jnp.float32(g))
        qlow = jnp.concatenate(lows, axis=0)
        qhigh = jnp.concatenate(highs, axis=0)
        ylow_ref[0, 0, hh] = qlow
        yhigh_ref[0, 0, hh] = qhigh
        for ai, a in enumerate(ALPHAS):
            ql = lows[ai]
            qh2 = highs[ai]
            c = jnp.float32(2.0 / a)
            pen = (jnp.abs(qh2 - ql)
                   + jnp.where(yrow < ql, (ql - yrow) * c, 0.0)
                   + jnp.where(yrow > qh2, (yrow - qh2) * c, 0.0))
            total = total + jnp.sum(pen)

    @pl.when(b == 0)
    def _init():
        sc_ref[...] = jnp.zeros_like(sc_ref)

    sc_ref[...] += total * jnp.float32(1.0 / (NHEADS * A * B * S))


def kernel(X_d, Y_d,
           h0_W0, h0_b0, h0_W1, h0_b1, h0_W2, h0_b2, h0_W3, h0_b3, hop0_Wq, hop0_Wk,
           h1_W0, h1_b0, h1_W1, h1_b1, h1_W2, h1_b2, h1_W3, h1_b3, hop1_Wq, hop1_Wk,
           h2_W0, h2_b0, h2_W1, h2_b1, h2_W2, h2_b2, h2_W3, h2_b3, hop2_Wq, hop2_Wk,
           h3_W0, h3_b0, h3_W1, h3_b1, h3_W2, h3_b2, h3_W3, h3_b3, hop3_Wq, hop3_Wk,
           train=0):
    per_model = [
        [h0_W0, h0_b0, h0_W1, h0_b1, h0_W2, h0_b2, h0_W3, h0_b3, hop0_Wq, hop0_Wk],
        [h1_W0, h1_b0, h1_W1, h1_b1, h1_W2, h1_b2, h1_W3, h1_b3, hop1_Wq, hop1_Wk],
        [h2_W0, h2_b0, h2_W1, h2_b1, h2_W2, h2_b2, h2_W3, h2_b3, hop2_Wq, hop2_Wk],
        [h3_W0, h3_b0, h3_W1, h3_b1, h3_W2, h3_b2, h3_W3, h3_b3, hop3_Wq, hop3_Wk],
    ]
    stacked = []
    for j in range(10):
        arrs = [per_model[i][j] for i in range(MHN)]
        st = jnp.stack(arrs)
        if st.ndim == 2:
            st = st[:, None, :]
        stacked.append(st)

    Y2 = Y_d[:, :, :, 0].reshape(B, MHN, 1, S)

    in_specs = [
        pl.BlockSpec((1, 1, S, CIN), lambda i, b: (b, i, 0, 0)),
        pl.BlockSpec((1, 1, 1, S), lambda i, b: (b, i, 0, 0)),
    ]
    for st in stacked:
        in_specs.append(
            pl.BlockSpec((1,) + st.shape[1:], lambda i, b: (i, 0, 0)))

    out_shape = [
        jax.ShapeDtypeStruct((B, MHN, NHEADS, A, S), jnp.float32),
        jax.ShapeDtypeStruct((B, MHN, NHEADS, A, S), jnp.float32),
        jax.ShapeDtypeStruct((MHN, 8, 128), jnp.float32),
    ]
    out_specs = [
        pl.BlockSpec((1, 1, NHEADS, A, S), lambda i, b: (b, i, 0, 0, 0)),
        pl.BlockSpec((1, 1, NHEADS, A, S), lambda i, b: (b, i, 0, 0, 0)),
        pl.BlockSpec((1, 8, 128), lambda i, b: (i, 0, 0)),
    ]

    ylow_k, yhigh_k, sc_k = pl.pallas_call(
        _body,
        grid=(MHN, B),
        in_specs=in_specs,
        out_specs=out_specs,
        out_shape=out_shape,
        compiler_params=pltpu.CompilerParams(
            dimension_semantics=("arbitrary", "arbitrary")),
    )(X_d, Y2, *stacked)

    scores = sc_k[:, 0, 0] + jnp.asarray(train, jnp.float32) * 0.0
    y = jnp.transpose(Y_d[:, :, :, 0], (0, 2, 1))
    y_low = jnp.transpose(ylow_k, (2, 0, 3, 4, 1))
    y_high = jnp.transpose(yhigh_k, (2, 0, 3, 4, 1))
    return (scores, y, y_low, y_high)
